# Initial kernel scaffold; baseline (speedup 1.0000x reference)
import jax
import jax.numpy as jnp
from jax.experimental import pallas as pl


def kernel(point_seq):
    raise NotImplementedError("write your pallas kernel here")


# speedup vs baseline: 3.4027x; 3.4027x over previous
import functools
import jax
import jax.numpy as jnp
from jax import lax
from jax.experimental import pallas as pl

K = 16
QT = 512


def _knn_step_kernel(q_ref, xt_ref, dist_ref, idx_ref):
    qx = q_ref[:, 0:1]
    qy = q_ref[:, 1:2]
    qz = q_ref[:, 2:3]
    kx = xt_ref[0:1, :]
    ky = xt_ref[1:2, :]
    kz = xt_ref[2:3, :]
    sq_q = (qx * qx + qy * qy) + qz * qz
    sq_k = (kx * kx + ky * ky) + kz * kz
    bf = jnp.bfloat16
    f32 = jnp.float32
    qxb, qyb, qzb = (v.astype(bf).astype(f32) for v in (qx, qy, qz))
    kxb, kyb, kzb = (v.astype(bf).astype(f32) for v in (kx, ky, kz))
    dot = (qxb * kxb + qyb * kyb) + qzb * kzb
    d = (sq_q + sq_k) - 2.0 * dot
    m = d.shape[1]
    iota = lax.broadcasted_iota(jnp.int32, (QT, m), 1)
    big = jnp.int32(2 ** 30)
    for j in range(K):
        mv = jnp.min(d, axis=1)
        sel = d == mv[:, None]
        ji = jnp.min(jnp.where(sel, iota, big), axis=1)
        dist_ref[:, j] = mv
        idx_ref[:, j] = ji
        if j + 1 < K:
            d = jnp.where(iota == ji[:, None], jnp.inf, d)


def _knn_step(q, xt):
    n = q.shape[0]
    m = xt.shape[1]
    grid = (n // QT,)
    return pl.pallas_call(
        _knn_step_kernel,
        grid=grid,
        in_specs=[
            pl.BlockSpec((QT, 3), lambda i: (i, 0)),
            pl.BlockSpec((3, m), lambda i: (0, 0)),
        ],
        out_specs=[
            pl.BlockSpec((QT, K), lambda i: (i, 0)),
            pl.BlockSpec((QT, K), lambda i: (i, 0)),
        ],
        out_shape=[
            jax.ShapeDtypeStruct((n, K), jnp.float32),
            jax.ShapeDtypeStruct((n, K), jnp.int32),
        ],
    )(q, xt)


def kernel(point_seq):
    b, t, n, d = point_seq.shape
    x2 = jnp.concatenate([point_seq[:, :1], point_seq], axis=1)[:, :-1]
    x_current = point_seq[:, 0]
    step = jax.vmap(_knn_step)
    dist_l, idx_l, pp_l = [], [], []
    for i in range(t):
        x_next = x2[:, i]
        xt = jnp.transpose(x_next, (0, 2, 1))
        dist, idx = step(x_current, xt)
        gathered = jax.vmap(lambda p, ii: p[ii])(x_next, idx)
        x_current = gathered[:, :, 0, :]
        dist_l.append(dist)
        idx_l.append(idx)
        pp_l.append(gathered)
    distances = jnp.stack(dist_l, axis=1)
    idxs = jnp.stack(idx_l, axis=1)
    patchlet_points = jnp.stack(pp_l, axis=1)
    out_x = patchlet_points[:, :, :, 0, :]
    anchor = patchlet_points[:, 0, :, 0:1, :][:, None]
    normalized = patchlet_points - anchor
    patchlet_feats = jnp.concatenate([patchlet_points, normalized], axis=-1)
    return (idxs, distances, idxs, patchlet_points, patchlet_feats,
            normalized, out_x)


# speedup vs baseline: 3.7598x; 1.1049x over previous
import functools
import jax
import jax.numpy as jnp
from jax import lax
from jax.experimental import pallas as pl

K = 16
NSLOT = 16
CT = 256
N = 4096


def _ce(m, va, ia, vb, ib):
    lo_v = jnp.where(m, va, vb)
    lo_i = jnp.where(m, ia, ib)
    hi_v = jnp.where(m, vb, va)
    hi_i = jnp.where(m, ib, ia)
    return lo_v, lo_i, hi_v, hi_i


def _ce_val(va, ia, vb, ib):
    return _ce(va <= vb, va, ia, vb, ib)


def _ce_lex(va, ia, vb, ib):
    m = (va < vb) | ((va == vb) & (ia < ib))
    return _ce(m, va, ia, vb, ib)


def _bitonic_merge(v, i, ce):
    n = len(v)
    d = n // 2
    while d >= 1:
        for start in range(0, n, 2 * d):
            for p in range(start, start + d):
                v[p], i[p], v[p + d], i[p + d] = ce(v[p], i[p], v[p + d], i[p + d])
        d //= 2
    return v, i


def _merge_sort(v, i, ce):
    n = len(v)
    if n == 1:
        return v, i
    h = n // 2
    av, ai = _merge_sort(v[:h], i[:h], ce)
    bv, bi = _merge_sort(v[h:], i[h:], ce)
    mv = av + bv[::-1]
    mi = ai + bi[::-1]
    return _bitonic_merge(mv, mi, ce)


def _knn_step_kernel(x_ref, qt_ref, dist_ref, idx_ref):
    f32 = jnp.float32
    bf = jnp.bfloat16
    qx = qt_ref[0:1, :]
    qy = qt_ref[1:2, :]
    qz = qt_ref[2:3, :]
    sq_q = (qx * qx + qy * qy) + qz * qz
    qxb, qyb, qzb = (v.astype(bf).astype(f32) for v in (qx, qy, qz))
    G = N // NSLOT
    vals, idxs = [], []
    giota = lax.broadcasted_iota(jnp.int32, (G, CT), 0)
    for j in range(NSLOT):
        kx = x_ref[j * G:(j + 1) * G, 0:1]
        ky = x_ref[j * G:(j + 1) * G, 1:2]
        kz = x_ref[j * G:(j + 1) * G, 2:3]
        sq_k = (kx * kx + ky * ky) + kz * kz
        kxb, kyb, kzb = (v.astype(bf).astype(f32) for v in (kx, ky, kz))
        dot = (kxb * qxb + kyb * qyb) + kzb * qzb
        vals.append((sq_q + sq_k) - 2.0 * dot)
        idxs.append(j * G + giota)

    vals, idxs = _merge_sort(vals, idxs, _ce_val)

    rows = G
    while rows > 1:
        h = rows // 2
        av = [x[:h] for x in vals]
        ai = [x[:h] for x in idxs]
        bv = [x[h:] for x in vals]
        bi = [x[h:] for x in idxs]
        nv, ni = [], []
        for p in range(NSLOT):
            m = av[p] <= bv[NSLOT - 1 - p]
            nv.append(jnp.where(m, av[p], bv[NSLOT - 1 - p]))
            ni.append(jnp.where(m, ai[p], bi[NSLOT - 1 - p]))
        vals, idxs = _bitonic_merge(nv, ni, _ce_val)
        rows = h

    vals, idxs = _merge_sort(vals, idxs, _ce_lex)

    dist_ref[:, :] = jnp.concatenate(vals, axis=0)
    idx_ref[:, :] = jnp.concatenate(idxs, axis=0)


def _knn_step(x, qt):
    grid = (N // CT,)
    return pl.pallas_call(
        _knn_step_kernel,
        grid=grid,
        in_specs=[
            pl.BlockSpec((N, 3), lambda i: (0, 0)),
            pl.BlockSpec((3, CT), lambda i: (0, i)),
        ],
        out_specs=[
            pl.BlockSpec((K, CT), lambda i: (0, i)),
            pl.BlockSpec((K, CT), lambda i: (0, i)),
        ],
        out_shape=[
            jax.ShapeDtypeStruct((K, N), jnp.float32),
            jax.ShapeDtypeStruct((K, N), jnp.int32),
        ],
    )(x, qt)


def kernel(point_seq):
    b, t, n, d = point_seq.shape
    x2 = jnp.concatenate([point_seq[:, :1], point_seq], axis=1)[:, :-1]
    x_current = point_seq[:, 0]
    step = jax.vmap(_knn_step)
    dist_l, idx_l, pp_l = [], [], []
    for i in range(t):
        x_next = x2[:, i]
        qt = jnp.transpose(x_current, (0, 2, 1))
        dist_t, idx_t = step(x_next, qt)
        dist = jnp.transpose(dist_t, (0, 2, 1))
        idx = jnp.transpose(idx_t, (0, 2, 1))
        gathered = jax.vmap(lambda p, ii: p[ii])(x_next, idx)
        x_current = gathered[:, :, 0, :]
        dist_l.append(dist)
        idx_l.append(idx)
        pp_l.append(gathered)
    distances = jnp.stack(dist_l, axis=1)
    idxs = jnp.stack(idx_l, axis=1)
    patchlet_points = jnp.stack(pp_l, axis=1)
    out_x = patchlet_points[:, :, :, 0, :]
    anchor = patchlet_points[:, 0, :, 0:1, :][:, None]
    normalized = patchlet_points - anchor
    patchlet_feats = jnp.concatenate([patchlet_points, normalized], axis=-1)
    return (idxs, distances, idxs, patchlet_points, patchlet_feats,
            normalized, out_x)


# speedup vs baseline: 11.7708x; 3.1307x over previous
import functools
import jax
import jax.numpy as jnp
from jax import lax
from jax.experimental import pallas as pl

K = 16
NSLOT = 16
CT = 128
N = 4096


def _ce(m, a, b):
    lo = [jnp.where(m, x, y) for x, y in zip(a, b)]
    hi = [jnp.where(m, y, x) for x, y in zip(a, b)]
    return lo, hi


def _ce_val(a, b):
    return _ce(a[0] <= b[0], a, b)


def _ce_lex(a, b):
    m = (a[0] < b[0]) | ((a[0] == b[0]) & (a[1] < b[1]))
    return _ce(m, a, b)


def _bitonic_merge(items, ce):
    n = len(items)
    d = n // 2
    while d >= 1:
        for start in range(0, n, 2 * d):
            for p in range(start, start + d):
                items[p], items[p + d] = ce(items[p], items[p + d])
        d //= 2
    return items


def _merge_sort(items, ce):
    n = len(items)
    if n == 1:
        return items
    h = n // 2
    a = _merge_sort(items[:h], ce)
    b = _merge_sort(items[h:], ce)
    return _bitonic_merge(a + b[::-1], ce)


def _knn_step_kernel(x_ref, qt_ref, dist_ref, idx_ref, px_ref, py_ref, pz_ref):
    f32 = jnp.float32
    bf = jnp.bfloat16
    qx = qt_ref[0:1, :]
    qy = qt_ref[1:2, :]
    qz = qt_ref[2:3, :]
    sq_q = (qx * qx + qy * qy) + qz * qz
    qxb, qyb, qzb = (v.astype(bf).astype(f32) for v in (qx, qy, qz))
    G = N // NSLOT
    giota = lax.broadcasted_iota(jnp.int32, (G, CT), 0)
    items = []
    for j in range(NSLOT):
        kx = x_ref[j * G:(j + 1) * G, 0:1]
        ky = x_ref[j * G:(j + 1) * G, 1:2]
        kz = x_ref[j * G:(j + 1) * G, 2:3]
        sq_k = (kx * kx + ky * ky) + kz * kz
        kxb, kyb, kzb = (v.astype(bf).astype(f32) for v in (kx, ky, kz))
        dot = (kxb * qxb + kyb * qyb) + kzb * qzb
        d = (sq_q + sq_k) - 2.0 * dot
        ones = jnp.ones((G, CT), f32)
        items.append((d, j * G + giota, kx * ones, ky * ones, kz * ones))

    items = _merge_sort(items, _ce_val)

    rows = G
    while rows > 1:
        h = rows // 2
        a = [tuple(x[:h] for x in it) for it in items]
        b = [tuple(x[h:] for x in it) for it in items]
        nitems = []
        for p in range(NSLOT):
            bb = b[NSLOT - 1 - p]
            m = a[p][0] <= bb[0]
            nitems.append(tuple(jnp.where(m, x, y) for x, y in zip(a[p], bb)))
        items = _bitonic_merge(nitems, _ce_val)
        rows = h

    items = _merge_sort(items, _ce_lex)

    dist_ref[:, :] = jnp.concatenate([it[0] for it in items], axis=0)
    idx_ref[:, :] = jnp.concatenate([it[1] for it in items], axis=0)
    px_ref[:, :] = jnp.concatenate([it[2] for it in items], axis=0)
    py_ref[:, :] = jnp.concatenate([it[3] for it in items], axis=0)
    pz_ref[:, :] = jnp.concatenate([it[4] for it in items], axis=0)


def _knn_step(x, qt):
    grid = (N // CT,)
    f32 = jnp.float32
    return pl.pallas_call(
        _knn_step_kernel,
        grid=grid,
        in_specs=[
            pl.BlockSpec((N, 3), lambda i: (0, 0)),
            pl.BlockSpec((3, CT), lambda i: (0, i)),
        ],
        out_specs=[pl.BlockSpec((K, CT), lambda i: (0, i))] * 5,
        out_shape=[
            jax.ShapeDtypeStruct((K, N), f32),
            jax.ShapeDtypeStruct((K, N), jnp.int32),
            jax.ShapeDtypeStruct((K, N), f32),
            jax.ShapeDtypeStruct((K, N), f32),
            jax.ShapeDtypeStruct((K, N), f32),
        ],
    )(x, qt)


def kernel(point_seq):
    b, t, n, d = point_seq.shape
    x2 = jnp.concatenate([point_seq[:, :1], point_seq], axis=1)[:, :-1]
    x_current = point_seq[:, 0]
    step = jax.vmap(_knn_step)
    dist_l, idx_l, pp_l = [], [], []
    for i in range(t):
        x_next = x2[:, i]
        qt = jnp.transpose(x_current, (0, 2, 1))
        dist_t, idx_t, px, py, pz = step(x_next, qt)
        dist = jnp.transpose(dist_t, (0, 2, 1))
        idx = jnp.transpose(idx_t, (0, 2, 1))
        gathered = jnp.stack([px, py, pz], axis=-1)
        gathered = jnp.transpose(gathered, (0, 2, 1, 3))
        x_current = gathered[:, :, 0, :]
        dist_l.append(dist)
        idx_l.append(idx)
        pp_l.append(gathered)
    distances = jnp.stack(dist_l, axis=1)
    idxs = jnp.stack(idx_l, axis=1)
    patchlet_points = jnp.stack(pp_l, axis=1)
    out_x = patchlet_points[:, :, :, 0, :]
    anchor = patchlet_points[:, 0, :, 0:1, :][:, None]
    normalized = patchlet_points - anchor
    patchlet_feats = jnp.concatenate([patchlet_points, normalized], axis=-1)
    return (idxs, distances, idxs, patchlet_points, patchlet_feats,
            normalized, out_x)


# speedup vs baseline: 13.0055x; 1.1049x over previous
import functools
import jax
import jax.numpy as jnp
from jax import lax
from jax.experimental import pallas as pl

K = 16
NSLOT = 16
CT = 256
N = 4096


def _ce(m, a, b):
    lo = [jnp.where(m, x, y) for x, y in zip(a, b)]
    hi = [jnp.where(m, y, x) for x, y in zip(a, b)]
    return lo, hi


def _ce_val(a, b):
    return _ce(a[0] <= b[0], a, b)


def _ce_lex(a, b):
    m = (a[0] < b[0]) | ((a[0] == b[0]) & (a[1] < b[1]))
    return _ce(m, a, b)


def _bitonic_merge(items, ce):
    n = len(items)
    d = n // 2
    while d >= 1:
        for start in range(0, n, 2 * d):
            for p in range(start, start + d):
                items[p], items[p + d] = ce(items[p], items[p + d])
        d //= 2
    return items


def _merge_sort(items, ce):
    n = len(items)
    if n == 1:
        return items
    h = n // 2
    a = _merge_sort(items[:h], ce)
    b = _merge_sort(items[h:], ce)
    return _bitonic_merge(a + b[::-1], ce)


def _knn_step_kernel(x_ref, qt_ref, dist_ref, idx_ref, px_ref, py_ref, pz_ref):
    f32 = jnp.float32
    bf = jnp.bfloat16
    qx = qt_ref[0:1, :]
    qy = qt_ref[1:2, :]
    qz = qt_ref[2:3, :]
    sq_q = (qx * qx + qy * qy) + qz * qz
    qxb, qyb, qzb = (v.astype(bf).astype(f32) for v in (qx, qy, qz))
    G = N // NSLOT
    giota = lax.broadcasted_iota(jnp.int32, (G, CT), 0)
    items = []
    for j in range(NSLOT):
        kx = x_ref[j * G:(j + 1) * G, 0:1]
        ky = x_ref[j * G:(j + 1) * G, 1:2]
        kz = x_ref[j * G:(j + 1) * G, 2:3]
        sq_k = (kx * kx + ky * ky) + kz * kz
        kxb, kyb, kzb = (v.astype(bf).astype(f32) for v in (kx, ky, kz))
        dot = (kxb * qxb + kyb * qyb) + kzb * qzb
        d = (sq_q + sq_k) - 2.0 * dot
        items.append((d, j * G + giota, kx, ky, kz))

    items = _merge_sort(items, _ce_val)

    rows = G
    while rows > 1:
        h = rows // 2
        a = [tuple(x[:h] for x in it) for it in items]
        b = [tuple(x[h:] for x in it) for it in items]
        nitems = []
        for p in range(NSLOT):
            bb = b[NSLOT - 1 - p]
            m = a[p][0] <= bb[0]
            nitems.append(tuple(jnp.where(m, x, y) for x, y in zip(a[p], bb)))
        items = _bitonic_merge(nitems, _ce_val)
        rows = h

    items = _merge_sort(items, _ce_lex)

    dist_ref[:, :] = jnp.concatenate([it[0] for it in items], axis=0)
    idx_ref[:, :] = jnp.concatenate([it[1] for it in items], axis=0)
    px_ref[:, :] = jnp.concatenate([it[2] for it in items], axis=0)
    py_ref[:, :] = jnp.concatenate([it[3] for it in items], axis=0)
    pz_ref[:, :] = jnp.concatenate([it[4] for it in items], axis=0)


def _knn_step(x, qt):
    grid = (N // CT,)
    f32 = jnp.float32
    return pl.pallas_call(
        _knn_step_kernel,
        grid=grid,
        in_specs=[
            pl.BlockSpec((N, 3), lambda i: (0, 0)),
            pl.BlockSpec((3, CT), lambda i: (0, i)),
        ],
        out_specs=[pl.BlockSpec((K, CT), lambda i: (0, i))] * 5,
        out_shape=[
            jax.ShapeDtypeStruct((K, N), f32),
            jax.ShapeDtypeStruct((K, N), jnp.int32),
            jax.ShapeDtypeStruct((K, N), f32),
            jax.ShapeDtypeStruct((K, N), f32),
            jax.ShapeDtypeStruct((K, N), f32),
        ],
    )(x, qt)


def kernel(point_seq):
    b, t, n, d = point_seq.shape
    x2 = jnp.concatenate([point_seq[:, :1], point_seq], axis=1)[:, :-1]
    x_current = point_seq[:, 0]
    step = jax.vmap(_knn_step)
    dist_l, idx_l, pp_l = [], [], []
    for i in range(t):
        x_next = x2[:, i]
        qt = jnp.transpose(x_current, (0, 2, 1))
        dist_t, idx_t, px, py, pz = step(x_next, qt)
        dist = jnp.transpose(dist_t, (0, 2, 1))
        idx = jnp.transpose(idx_t, (0, 2, 1))
        gathered = jnp.stack([px, py, pz], axis=-1)
        gathered = jnp.transpose(gathered, (0, 2, 1, 3))
        x_current = gathered[:, :, 0, :]
        dist_l.append(dist)
        idx_l.append(idx)
        pp_l.append(gathered)
    distances = jnp.stack(dist_l, axis=1)
    idxs = jnp.stack(idx_l, axis=1)
    patchlet_points = jnp.stack(pp_l, axis=1)
    out_x = patchlet_points[:, :, :, 0, :]
    anchor = patchlet_points[:, 0, :, 0:1, :][:, None]
    normalized = patchlet_points - anchor
    patchlet_feats = jnp.concatenate([patchlet_points, normalized], axis=-1)
    return (idxs, distances, idxs, patchlet_points, patchlet_feats,
            normalized, out_x)
